# Initial kernel scaffold; baseline (speedup 1.0000x reference)
import jax
import jax.numpy as jnp
from jax.experimental import pallas as pl


def kernel(tok_emb, pos_emb, enc_w, enc_b, enc_g, enc_beta, mlm_w, mlm_b, act_w, act_b, seq, seq_mask, seq_lengths, labels, isnext):
    raise NotImplementedError("write your pallas kernel here")


# speedup vs baseline: 1.1205x; 1.1205x over previous
import functools

import jax
import jax.numpy as jnp
from jax.experimental import pallas as pl
from jax.experimental.pallas import tpu as pltpu


_VMEM_LIMIT = 100 * 1024 * 1024


def _cdiv(a, b):
    return (a + b - 1) // b


def _enc_kernel(x_ref, w_ref, b_ref, g_ref, beta_ref, o_ref):
    h = jnp.dot(x_ref[...], w_ref[...], preferred_element_type=jnp.float32)
    h = h + b_ref[...]
    h = 0.5 * h * (1.0 + jnp.tanh(0.7978845608028654 * (h + 0.044715 * h * h * h)))
    mu = jnp.mean(h, axis=-1, keepdims=True)
    var = jnp.mean((h - mu) ** 2, axis=-1, keepdims=True)
    out = (h - mu) * jax.lax.rsqrt(var + 1e-12) * g_ref[...] + beta_ref[...]
    o_ref[...] = out.astype(o_ref.dtype)


def _encoder_layer(x, w, b, g, beta, *, tile_rows=512):
    n, hdim = x.shape
    tn = min(tile_rows, n)
    return pl.pallas_call(
        _enc_kernel,
        out_shape=jax.ShapeDtypeStruct((n, hdim), jnp.bfloat16),
        grid=(_cdiv(n, tn),),
        in_specs=[
            pl.BlockSpec((tn, hdim), lambda i: (i, 0)),
            pl.BlockSpec((hdim, hdim), lambda i: (0, 0)),
            pl.BlockSpec((1, hdim), lambda i: (0, 0)),
            pl.BlockSpec((1, hdim), lambda i: (0, 0)),
            pl.BlockSpec((1, hdim), lambda i: (0, 0)),
        ],
        out_specs=pl.BlockSpec((tn, hdim), lambda i: (i, 0)),
        compiler_params=pltpu.CompilerParams(
            dimension_semantics=("parallel",),
            vmem_limit_bytes=_VMEM_LIMIT),
    )(x, w, b, g, beta)


def _mlm_kernel(x_ref, w_ref, b_ref, o_ref, m_sc, s_sc, *, tv):
    j = pl.program_id(1)

    @pl.when(j == 0)
    def _():
        m_sc[...] = jnp.full(m_sc.shape, -jnp.inf, m_sc.dtype)
        s_sc[...] = jnp.zeros(s_sc.shape, s_sc.dtype)

    logits = jnp.dot(x_ref[...], w_ref[...],
                     preferred_element_type=jnp.float32) + b_ref[...]
    o_ref[:, pl.ds(j * tv, tv)] = logits

    m_prev = m_sc[...]
    m_new = jnp.maximum(m_prev, jnp.max(logits, axis=-1, keepdims=True))
    s_sc[...] = (s_sc[...] * jnp.exp(m_prev - m_new)
                 + jnp.sum(jnp.exp(logits - m_new), axis=-1, keepdims=True))
    m_sc[...] = m_new

    @pl.when(j == pl.num_programs(1) - 1)
    def _():
        o_ref[...] = o_ref[...] - (m_sc[...] + jnp.log(s_sc[...]))


def _mlm_log_softmax(x, w, b, *, tile_rows=208, tile_v=2048):
    n, hdim = x.shape
    v = w.shape[1]
    tn = min(tile_rows, n)
    tv = min(tile_v, v)

    return pl.pallas_call(
        functools.partial(_mlm_kernel, tv=tv),
        out_shape=jax.ShapeDtypeStruct((n, v), jnp.float32),
        grid=(_cdiv(n, tn), _cdiv(v, tv)),
        in_specs=[
            pl.BlockSpec((tn, hdim), lambda i, j: (i, 0)),
            pl.BlockSpec((hdim, tv), lambda i, j: (0, j)),
            pl.BlockSpec((1, tv), lambda i, j: (0, j)),
        ],
        out_specs=pl.BlockSpec((tn, v), lambda i, j: (i, 0)),
        scratch_shapes=[pltpu.VMEM((tn, 1), jnp.float32),
                        pltpu.VMEM((tn, 1), jnp.float32)],
        compiler_params=pltpu.CompilerParams(
            dimension_semantics=("parallel", "arbitrary"),
            vmem_limit_bytes=_VMEM_LIMIT),
    )(x, w, b)


@jax.jit
def _forward(tok_emb, pos_emb, enc_w, enc_b, enc_g, enc_beta, mlm_w, mlm_b,
             act_w, act_b, seq, seq_mask, isnext):
    B, L = seq.shape
    H = tok_emb.shape[1]
    V = mlm_w.shape[1]

    emb = tok_emb[seq] + pos_emb[:L][None]
    x = (emb * seq_mask[..., None]).astype(jnp.bfloat16)

    ctx = _encoder_layer(
        x.reshape(B * L, H), enc_w, enc_b, enc_g, enc_beta,
    ).reshape(B, L, H)

    cls_part = ctx[:, 0, :]
    lang_part = ctx[:, 1:, :]
    l_lang = L - 1

    mask_lm_output = _mlm_log_softmax(
        lang_part.reshape(B * l_lang, H), mlm_w, mlm_b,
    ).reshape(B, l_lang, V)

    logits_a = cls_part.astype(jnp.float32) @ act_w + act_b
    next_action_output = jax.nn.log_softmax(logits_a, axis=-1)

    tgt = isnext.astype(jnp.int32)
    valid = tgt != 0
    safe = jnp.clip(tgt, 0, next_action_output.shape[-1] - 1)
    picked = jnp.take_along_axis(next_action_output, safe[:, None], axis=-1)[:, 0]
    n_valid = jnp.sum(valid.astype(jnp.float32))
    loss = jnp.sum(jnp.where(valid, -picked, 0.0)) / jnp.maximum(n_valid, 1.0)

    return next_action_output, mask_lm_output, loss


def kernel(tok_emb, pos_emb, enc_w, enc_b, enc_g, enc_beta, mlm_w, mlm_b,
           act_w, act_b, seq, seq_mask, seq_lengths, labels, isnext):
    return _forward(tok_emb, pos_emb, enc_w, enc_b, enc_g, enc_beta,
                    mlm_w, mlm_b, act_w, act_b, seq, seq_mask, isnext)
